# baseline (speedup 1.0000x reference)
import jax
import jax.numpy as jnp
from jax.experimental import pallas as pl
from jax.experimental.pallas import tpu as pltpu

T, DY, H = 8192, 256, 1024
R_W1Y, R_W2, R_B2, R_W3, R_B3 = 8192, 8448, 9472, 9480, 10504
DT = 100.0
SLOTS = (0, 0, 0, 0, 1, 1, 1, 1, 1, 1, 1, 1, 1, 2, 2, 2)
NUM_OFF = 3
TILE_B = 512
N_STEPS = 16


def _euler_kernel(idx_sref, y0_ref, slab_ref, w1y_ref, w2_ref, w3_ref,
                  out_ref, g0, g1, g2):
    i = pl.program_id(0)
    g = (g0, g1, g2)

    for mi in range(TILE_B):
        b = idx_sref[i * TILE_B + mi]
        for o in range(NUM_OFF):
            r = jnp.minimum(b + o, T - 1) if o else b
            c8 = pl.multiple_of((r >> 3) << 3, 8)
            chunk = slab_ref[pl.ds(c8, 8), :]
            row = pltpu.roll(chunk, -(r & 7), axis=0)[0:1, :]
            g[o][mi:mi + 1, :] = row

    hx = [go[...] for go in g]

    w1y = w1y_ref[...]
    w2 = w2_ref[...]
    w3 = w3_ref[...]
    b2 = slab_ref[R_B2:R_B2 + 1, :]
    b3 = slab_ref[R_B3:R_B3 + 1, :DY]

    y = y0_ref[...]
    for k in range(N_STEPS):
        h1 = jnp.tanh(hx[SLOTS[k]]
                      + jnp.dot(y.astype(jnp.bfloat16), w1y,
                                preferred_element_type=jnp.float32))
        h2 = jnp.tanh(jnp.dot(h1.astype(jnp.bfloat16), w2,
                              preferred_element_type=jnp.float32) + b2)
        y = y + DT * (jnp.dot(h2.astype(jnp.bfloat16), w3,
                              preferred_element_type=jnp.float32) + b3)

    out_ref[...] = y


def kernel(y0, base_idx, slab):
    batch, dy = y0.shape
    assert dy == DY
    idx = base_idx.astype(jnp.int32)
    w1y = slab[R_W1Y:R_W1Y + DY, :].astype(jnp.bfloat16)
    w2 = slab[R_W2:R_W2 + H, :].astype(jnp.bfloat16)
    w3 = slab[R_W3:R_W3 + H, :DY].astype(jnp.bfloat16)

    out = pl.pallas_call(
        _euler_kernel,
        out_shape=jax.ShapeDtypeStruct((batch, DY), jnp.float32),
        grid_spec=pltpu.PrefetchScalarGridSpec(
            num_scalar_prefetch=1,
            grid=(batch // TILE_B,),
            in_specs=[
                pl.BlockSpec((TILE_B, DY), lambda i, idxs: (i, 0)),
                pl.BlockSpec(slab.shape, lambda i, idxs: (0, 0)),
                pl.BlockSpec(w1y.shape, lambda i, idxs: (0, 0)),
                pl.BlockSpec(w2.shape, lambda i, idxs: (0, 0)),
                pl.BlockSpec(w3.shape, lambda i, idxs: (0, 0)),
            ],
            out_specs=pl.BlockSpec((TILE_B, DY), lambda i, idxs: (i, 0)),
            scratch_shapes=[pltpu.VMEM((TILE_B, H), jnp.float32)
                            for _ in range(NUM_OFF)],
        ),
        compiler_params=pltpu.CompilerParams(
            dimension_semantics=("parallel",)),
    )(idx, y0, slab, w1y, w2, w3)
    return out


# speedup vs baseline: 1.1730x; 1.1730x over previous
import jax
import jax.numpy as jnp
from jax.experimental import pallas as pl
from jax.experimental.pallas import tpu as pltpu

T, DY, H = 8192, 256, 1024
R_W1Y, R_W2, R_B2, R_W3, R_B3 = 8192, 8448, 9472, 9480, 10504
DT = 100.0
SLOTS = (0, 0, 0, 0, 1, 1, 1, 1, 1, 1, 1, 1, 1, 2, 2, 2)
NUM_OFF = 3
TILE_B = 512
N_STEPS = 16


def _euler_kernel(idx_sref, y0_ref, slab_ref, out_ref, g0, g1, g2):
    i = pl.program_id(0)
    g = (g0, g1, g2)

    for o in range(NUM_OFF):
        for mi in range(TILE_B):
            b = idx_sref[i * TILE_B + mi]
            r = jnp.minimum(b + o, T - 1) if o else b
            c8 = pl.multiple_of((r >> 3) << 3, 8)
            chunk = slab_ref[pl.ds(c8, 8), :]
            row = pltpu.roll(chunk, -(r & 7), axis=0)[0:1, :]
            g[o][mi:mi + 1, :] = row

    hx = [go[...] for go in g]

    w1y = slab_ref[R_W1Y:R_W1Y + DY, :]
    w2 = slab_ref[R_W2:R_W2 + H, :]
    b2 = slab_ref[R_B2:R_B2 + 1, :]
    w3 = slab_ref[R_W3:R_W3 + H, :DY]
    b3 = slab_ref[R_B3:R_B3 + 1, :DY]

    y = y0_ref[...]
    for k in range(N_STEPS):
        h1 = jnp.tanh(hx[SLOTS[k]]
                      + jnp.dot(y, w1y, preferred_element_type=jnp.float32))
        h2 = jnp.tanh(jnp.dot(h1, w2, preferred_element_type=jnp.float32) + b2)
        y = y + DT * (jnp.dot(h2, w3, preferred_element_type=jnp.float32) + b3)

    out_ref[...] = y


def kernel(y0, base_idx, slab):
    batch, dy = y0.shape
    assert dy == DY
    idx = base_idx.astype(jnp.int32)

    out = pl.pallas_call(
        _euler_kernel,
        out_shape=jax.ShapeDtypeStruct((batch, DY), jnp.float32),
        grid_spec=pltpu.PrefetchScalarGridSpec(
            num_scalar_prefetch=1,
            grid=(batch // TILE_B,),
            in_specs=[
                pl.BlockSpec((TILE_B, DY), lambda i, idxs: (i, 0)),
                pl.BlockSpec(slab.shape, lambda i, idxs: (0, 0)),
            ],
            out_specs=pl.BlockSpec((TILE_B, DY), lambda i, idxs: (i, 0)),
            scratch_shapes=[pltpu.VMEM((TILE_B, H), jnp.float32)
                            for _ in range(NUM_OFF)],
        ),
        compiler_params=pltpu.CompilerParams(
            dimension_semantics=("parallel",)),
    )(idx, y0, slab)
    return out


# speedup vs baseline: 1.1976x; 1.0210x over previous
import jax
import jax.numpy as jnp
from jax.experimental import pallas as pl
from jax.experimental.pallas import tpu as pltpu

T, DY, H = 8192, 256, 1024
R_W1Y, R_W2, R_B2, R_W3, R_B3 = 8192, 8448, 9472, 9480, 10504
DT = 100.0
SLOTS = (0, 0, 0, 0, 1, 1, 1, 1, 1, 1, 1, 1, 1, 2, 2, 2)
NUM_OFF = 3
TILE_B = 512
N_STEPS = 16


def _euler_kernel(idx_sref, y0_ref, slab_ref, out_ref, g0, g1, g2):
    i = pl.program_id(0)
    g = (g0, g1, g2)

    for mi in range(TILE_B):
        b = idx_sref[i * TILE_B + mi]
        for o in range(NUM_OFF):
            r = jnp.minimum(b + o, T - 1) if o else b
            c8 = pl.multiple_of((r >> 3) << 3, 8)
            chunk = slab_ref[pl.ds(c8, 8), :]
            row = pltpu.roll(chunk, -(r & 7), axis=0)[0:1, :]
            g[o][mi:mi + 1, :] = row

    w1y = slab_ref[R_W1Y:R_W1Y + DY, :]
    w2 = slab_ref[R_W2:R_W2 + H, :]
    b2 = slab_ref[R_B2:R_B2 + 1, :]
    w3 = slab_ref[R_W3:R_W3 + H, :DY]
    b3 = slab_ref[R_B3:R_B3 + 1, :DY]

    y = y0_ref[...]
    for k in range(N_STEPS):
        h1 = jnp.tanh(g[SLOTS[k]][...]
                      + jnp.dot(y, w1y, preferred_element_type=jnp.float32))
        h2 = jnp.tanh(jnp.dot(h1, w2, preferred_element_type=jnp.float32) + b2)
        y = y + DT * (jnp.dot(h2, w3, preferred_element_type=jnp.float32) + b3)

    out_ref[...] = y


def kernel(y0, base_idx, slab):
    batch, dy = y0.shape
    assert dy == DY
    idx = base_idx.astype(jnp.int32)

    out = pl.pallas_call(
        _euler_kernel,
        out_shape=jax.ShapeDtypeStruct((batch, DY), jnp.float32),
        grid_spec=pltpu.PrefetchScalarGridSpec(
            num_scalar_prefetch=1,
            grid=(batch // TILE_B,),
            in_specs=[
                pl.BlockSpec((TILE_B, DY), lambda i, idxs: (i, 0)),
                pl.BlockSpec(slab.shape, lambda i, idxs: (0, 0)),
            ],
            out_specs=pl.BlockSpec((TILE_B, DY), lambda i, idxs: (i, 0)),
            scratch_shapes=[pltpu.VMEM((TILE_B, H), jnp.float32)
                            for _ in range(NUM_OFF)],
        ),
        compiler_params=pltpu.CompilerParams(
            dimension_semantics=("parallel",)),
    )(idx, y0, slab)
    return out
